# baseline (speedup 1.0000x reference)
import jax
import jax.numpy as jnp
from jax.experimental import pallas as pl

_B, _S, _DIN, _E, _G, _C = 16, 4096, 8, 64, 128, 8
_R = _B * _DIN
_CH2 = 512
_CH3 = 256
_PAD = 0.0


def _stats_kernel(x2_ref, lrep_ref, mean_ref, sv_ref):
    f32 = jnp.float32
    x2 = x2_ref[...]
    lrep = lrep_ref[...]
    il = jax.lax.broadcasted_iota(jnp.int32, (_R, _S), 1)
    mf = (il < lrep).astype(f32)
    cnt = jnp.maximum(jnp.sum(mf) * (1.0 / _DIN), 1.0)

    rd = jax.lax.broadcasted_iota(jnp.int32, (_DIN, _R), 1)
    dd = jax.lax.broadcasted_iota(jnp.int32, (_DIN, _R), 0)
    m8 = (jax.lax.rem(rd, _DIN) == dd).astype(f32)
    rr = jax.lax.broadcasted_iota(jnp.int32, (_R, _DIN), 0)
    dc = jax.lax.broadcasted_iota(jnp.int32, (_R, _DIN), 1)
    p8 = (jax.lax.rem(rr, _DIN) == dc).astype(f32)

    hi = jax.lax.Precision.HIGHEST
    sum_rows = jnp.sum(x2 * mf, axis=1, keepdims=True)
    dn = (((1,), (0,)), ((), ()))
    mean8 = jax.lax.dot_general(m8, sum_rows, dn, precision=hi) / cnt
    mean_r = jax.lax.dot_general(p8, mean8, dn, precision=hi)
    xc = x2 - mean_r
    sq_rows = jnp.sum((xc * xc) * mf, axis=1, keepdims=True)
    var8 = jax.lax.dot_general(m8, sq_rows, dn, precision=hi) / cnt
    sv8 = jnp.sqrt(var8 + 1e-6)

    i8r = jax.lax.broadcasted_iota(jnp.int32, (_DIN, _DIN), 0)
    i8c = jax.lax.broadcasted_iota(jnp.int32, (_DIN, _DIN), 1)
    eye8 = (i8r == i8c).astype(f32)
    dt = (((0,), (0,)), ((), ()))
    mean_ref[...] = jax.lax.dot_general(mean8, eye8, dt, precision=hi)
    sv_ref[...] = jax.lax.dot_general(sv8, eye8, dt, precision=hi)


def _mlp_h(xs, mean_row, sv_row, w1, b1r, w2, b2r):
    xn = (xs - mean_row) / sv_row
    h = jnp.maximum(xn @ w1 + b1r, 0.0)
    return jnp.maximum(h @ w2 + b2r, 0.0)


def _rows_from_packed(x2):
    return jnp.concatenate(
        [jnp.transpose(x2[_DIN * b:_DIN * (b + 1), :]) for b in range(_B)],
        axis=0)


def _gfv_kernel(x2_ref, len3_ref, mean_ref, sv_ref, w1_ref, b1r_ref,
                w2_ref, b2r_ref, wg_ref, bgr_ref, gfv_ref):
    k = pl.program_id(0)
    base = k * _CH2
    len3 = len3_ref[...]

    h = _mlp_h(_rows_from_packed(x2_ref[...]), mean_ref[...], sv_ref[...],
               w1_ref[...], b1r_ref[...], w2_ref[...], b2r_ref[...])
    g = jnp.maximum(h @ wg_ref[...] + bgr_ref[...], 0.0)

    ig = jax.lax.broadcasted_iota(jnp.int32, (_B, _CH2, _G), 1) + base
    maskg = ig < jnp.broadcast_to(len3, (_B, _CH2, _G))
    g3 = jnp.where(maskg, g.reshape(_B, _CH2, _G), -jnp.inf)
    part = jnp.max(g3, axis=1)

    @pl.when(k == 0)
    def _():
        gfv_ref[...] = jnp.full((_B, _G), -jnp.inf, jnp.float32)

    gfv_ref[...] = jnp.maximum(gfv_ref[...], part)


def _out_kernel(x2_ref, len3_ref, mean_ref, sv_ref, w1_ref, b1r_ref,
                w2_ref, b2r_ref, gfv3_ref, wseg_ref, bsegr_ref,
                logits_ref, labels_ref, scores_ref, cat_ref):
    k = pl.program_id(0)
    base = k * _CH3
    len3 = len3_ref[...]

    n = _B * _CH3
    h = _mlp_h(_rows_from_packed(x2_ref[...]), mean_ref[...], sv_ref[...],
               w1_ref[...], b1r_ref[...], w2_ref[...], b2r_ref[...])

    ie = jax.lax.broadcasted_iota(jnp.int32, (_B, _CH3, _E), 1) + base
    maske = ie < jnp.broadcast_to(len3, (_B, _CH3, _E))
    h3 = jnp.where(maske, h.reshape(_B, _CH3, _E), _PAD)

    ig = jax.lax.broadcasted_iota(jnp.int32, (_B, _CH3, _G), 1) + base
    maskg = ig < jnp.broadcast_to(len3, (_B, _CH3, _G))
    gfv3 = jnp.broadcast_to(gfv3_ref[...], (_B, _CH3, _G))
    gfv3 = jnp.where(maskg, gfv3, _PAD)

    cat3 = jnp.concatenate([h3, gfv3], axis=2)
    cat_ref[...] = cat3

    logits = cat3.reshape(n, _E + _G) @ wseg_ref[...] + bsegr_ref[...]
    ic = jax.lax.broadcasted_iota(jnp.int32, (_B, _CH3, _C), 1) + base
    maskc = ic < jnp.broadcast_to(len3, (_B, _CH3, _C))
    logits2 = jnp.where(maskc.reshape(n, _C), logits, _PAD)
    logits_ref[...] = logits2.reshape(_B, _CH3, _C)

    lt = jnp.transpose(logits2)
    m = jnp.max(lt, axis=0, keepdims=True)
    ssum = jnp.sum(jnp.exp(lt - m), axis=0, keepdims=True)
    scores = 1.0 / ssum
    scores_ref[...] = jnp.transpose(scores).reshape(_B, _CH3, 1)

    cidx = jax.lax.broadcasted_iota(jnp.int32, (_C, n), 0)
    cand = jnp.where(lt == m, cidx, _C)
    labels = jnp.min(cand, axis=0, keepdims=True)
    labels = jnp.where(jnp.isnan(scores), -1, labels)
    labels_ref[...] = jnp.transpose(labels).reshape(_B, _CH3, 1)


def kernel(x, lengths, W1, b1, W2, b2, Wg, bg, Wseg, bseg):
    f32 = jnp.float32
    x2 = x.transpose(0, 2, 1).reshape(_R, _S)
    lrep = jnp.repeat(lengths.astype(jnp.int32), _DIN).reshape(_R, 1)
    len3 = lengths.astype(jnp.int32).reshape(_B, 1, 1)
    b1r = b1.reshape(1, _E)
    b2r = b2.reshape(1, _E)
    bgr = bg.reshape(1, _G)
    bsegr = bseg.reshape(1, _C)

    mean_row, sv_row = pl.pallas_call(
        _stats_kernel,
        out_shape=[
            jax.ShapeDtypeStruct((1, _DIN), f32),
            jax.ShapeDtypeStruct((1, _DIN), f32),
        ],
    )(x2, lrep)

    n2 = _S // _CH2
    gfv = pl.pallas_call(
        _gfv_kernel,
        grid=(n2,),
        in_specs=[
            pl.BlockSpec((_R, _CH2), lambda k: (0, k)),
            pl.BlockSpec((_B, 1, 1), lambda k: (0, 0, 0)),
            pl.BlockSpec((1, _DIN), lambda k: (0, 0)),
            pl.BlockSpec((1, _DIN), lambda k: (0, 0)),
            pl.BlockSpec((_DIN, _E), lambda k: (0, 0)),
            pl.BlockSpec((1, _E), lambda k: (0, 0)),
            pl.BlockSpec((_E, _E), lambda k: (0, 0)),
            pl.BlockSpec((1, _E), lambda k: (0, 0)),
            pl.BlockSpec((_E, _G), lambda k: (0, 0)),
            pl.BlockSpec((1, _G), lambda k: (0, 0)),
        ],
        out_specs=pl.BlockSpec((_B, _G), lambda k: (0, 0)),
        out_shape=jax.ShapeDtypeStruct((_B, _G), f32),
    )(x2, len3, mean_row, sv_row, W1, b1r, W2, b2r, Wg, bgr)

    gfv3 = gfv.reshape(_B, 1, _G)

    n3 = _S // _CH3
    logits, labels, scores, cat = pl.pallas_call(
        _out_kernel,
        grid=(n3,),
        in_specs=[
            pl.BlockSpec((_R, _CH3), lambda k: (0, k)),
            pl.BlockSpec((_B, 1, 1), lambda k: (0, 0, 0)),
            pl.BlockSpec((1, _DIN), lambda k: (0, 0)),
            pl.BlockSpec((1, _DIN), lambda k: (0, 0)),
            pl.BlockSpec((_DIN, _E), lambda k: (0, 0)),
            pl.BlockSpec((1, _E), lambda k: (0, 0)),
            pl.BlockSpec((_E, _E), lambda k: (0, 0)),
            pl.BlockSpec((1, _E), lambda k: (0, 0)),
            pl.BlockSpec((_B, 1, _G), lambda k: (0, 0, 0)),
            pl.BlockSpec((_E + _G, _C), lambda k: (0, 0)),
            pl.BlockSpec((1, _C), lambda k: (0, 0)),
        ],
        out_specs=[
            pl.BlockSpec((_B, _CH3, _C), lambda k: (0, k, 0)),
            pl.BlockSpec((_B, _CH3, 1), lambda k: (0, k, 0)),
            pl.BlockSpec((_B, _CH3, 1), lambda k: (0, k, 0)),
            pl.BlockSpec((_B, _CH3, _E + _G), lambda k: (0, k, 0)),
        ],
        out_shape=[
            jax.ShapeDtypeStruct((_B, _S, _C), f32),
            jax.ShapeDtypeStruct((_B, _S, 1), jnp.int32),
            jax.ShapeDtypeStruct((_B, _S, 1), f32),
            jax.ShapeDtypeStruct((_B, _S, _E + _G), f32),
        ],
    )(x2, len3, mean_row, sv_row, W1, b1r, W2, b2r, gfv3, Wseg, bsegr)

    return (logits, labels, scores, cat)


# speedup vs baseline: 1.3792x; 1.3792x over previous
import jax
import jax.numpy as jnp
from jax.experimental import pallas as pl

_B, _S, _DIN, _E, _G, _C = 16, 4096, 8, 64, 128, 8
_R = _B * _DIN
_CH2 = 512
_CH3 = 256
_PAD = 0.0


def _stats_kernel(x2_ref, lrep_ref, mean_ref, sv_ref):
    f32 = jnp.float32
    x2 = x2_ref[...]
    lrep = lrep_ref[...]
    il = jax.lax.broadcasted_iota(jnp.int32, (_R, _S), 1)
    mf = (il < lrep).astype(f32)
    cnt = jnp.maximum(jnp.sum(mf) * (1.0 / _DIN), 1.0)

    rd = jax.lax.broadcasted_iota(jnp.int32, (_DIN, _R), 1)
    dd = jax.lax.broadcasted_iota(jnp.int32, (_DIN, _R), 0)
    m8 = (jax.lax.rem(rd, _DIN) == dd).astype(f32)
    rr = jax.lax.broadcasted_iota(jnp.int32, (_R, _DIN), 0)
    dc = jax.lax.broadcasted_iota(jnp.int32, (_R, _DIN), 1)
    p8 = (jax.lax.rem(rr, _DIN) == dc).astype(f32)

    hi = jax.lax.Precision.HIGHEST
    sum_rows = jnp.sum(x2 * mf, axis=1, keepdims=True)
    dn = (((1,), (0,)), ((), ()))
    mean8 = jax.lax.dot_general(m8, sum_rows, dn, precision=hi) / cnt
    mean_r = jax.lax.dot_general(p8, mean8, dn, precision=hi)
    xc = x2 - mean_r
    sq_rows = jnp.sum((xc * xc) * mf, axis=1, keepdims=True)
    var8 = jax.lax.dot_general(m8, sq_rows, dn, precision=hi) / cnt
    sv8 = jnp.sqrt(var8 + 1e-6)

    i8r = jax.lax.broadcasted_iota(jnp.int32, (_DIN, _DIN), 0)
    i8c = jax.lax.broadcasted_iota(jnp.int32, (_DIN, _DIN), 1)
    eye8 = (i8r == i8c).astype(f32)
    dt = (((0,), (0,)), ((), ()))
    mean_ref[...] = jax.lax.dot_general(mean8, eye8, dt, precision=hi)
    sv_ref[...] = jax.lax.dot_general(sv8, eye8, dt, precision=hi)


def _mlp_h(xs, mean_row, sv_row, w1, b1r, w2, b2r):
    xn = (xs - mean_row) / sv_row
    h = jnp.maximum(xn @ w1 + b1r, 0.0)
    return jnp.maximum(h @ w2 + b2r, 0.0)


def _rows_from_packed(x2):
    return jnp.concatenate(
        [jnp.transpose(x2[_DIN * b:_DIN * (b + 1), :]) for b in range(_B)],
        axis=0)


def _gfv_kernel(x2_ref, len3_ref, mean_ref, sv_ref, w1_ref, b1r_ref,
                w2_ref, b2r_ref, wg_ref, bgr_ref, gfv_ref):
    k = pl.program_id(0)
    base = k * _CH2
    len3 = len3_ref[...]

    h = _mlp_h(_rows_from_packed(x2_ref[...]), mean_ref[...], sv_ref[...],
               w1_ref[...], b1r_ref[...], w2_ref[...], b2r_ref[...])
    g = jnp.maximum(h @ wg_ref[...] + bgr_ref[...], 0.0)

    ig = jax.lax.broadcasted_iota(jnp.int32, (_B, _CH2, _G), 1) + base
    maskg = ig < jnp.broadcast_to(len3, (_B, _CH2, _G))
    g3 = jnp.where(maskg, g.reshape(_B, _CH2, _G), -jnp.inf)
    part = jnp.max(g3, axis=1)

    @pl.when(k == 0)
    def _():
        gfv_ref[...] = jnp.full((_B, _G), -jnp.inf, jnp.float32)

    gfv_ref[...] = jnp.maximum(gfv_ref[...], part)


def _out_kernel(x2_ref, len3_ref, mean_ref, sv_ref, w1_ref, b1r_ref,
                w2_ref, b2r_ref, gfv3_ref, wseg_ref, bsegr_ref,
                logits_ref, labels_ref, scores_ref, cat_ref):
    k = pl.program_id(0)
    base = k * _CH3
    len3 = len3_ref[...]

    n = _B * _CH3
    h = _mlp_h(_rows_from_packed(x2_ref[...]), mean_ref[...], sv_ref[...],
               w1_ref[...], b1r_ref[...], w2_ref[...], b2r_ref[...])

    ie = jax.lax.broadcasted_iota(jnp.int32, (_B, _CH3, _E), 1) + base
    maske = ie < jnp.broadcast_to(len3, (_B, _CH3, _E))
    h3 = jnp.where(maske, h.reshape(_B, _CH3, _E), _PAD)

    ig = jax.lax.broadcasted_iota(jnp.int32, (_B, _CH3, _G), 1) + base
    maskg = ig < jnp.broadcast_to(len3, (_B, _CH3, _G))
    gfv3 = jnp.broadcast_to(gfv3_ref[...], (_B, _CH3, _G))
    gfv3 = jnp.where(maskg, gfv3, _PAD)

    cat3 = jnp.concatenate([h3, gfv3], axis=2)
    cat_ref[...] = cat3

    logits = cat3.reshape(n, _E + _G) @ wseg_ref[...] + bsegr_ref[...]
    ic = jax.lax.broadcasted_iota(jnp.int32, (_B, _CH3, _C), 1) + base
    maskc = ic < jnp.broadcast_to(len3, (_B, _CH3, _C))
    logits2 = jnp.where(maskc.reshape(n, _C), logits, _PAD)
    logits_ref[...] = logits2.reshape(_B, _CH3, _C)

    lt = jnp.transpose(logits2)
    m = jnp.max(lt, axis=0, keepdims=True)
    ssum = jnp.sum(jnp.exp(lt - m), axis=0, keepdims=True)
    scores = 1.0 / ssum

    cidx = jax.lax.broadcasted_iota(jnp.int32, (_C, n), 0)
    cand = jnp.where(lt == m, cidx, _C)
    labels = jnp.min(cand, axis=0, keepdims=True)
    labels = jnp.where(jnp.isnan(scores), -1, labels)

    def _rows(v):
        return jnp.concatenate(
            [v[:, _CH3 * b:_CH3 * (b + 1)] for b in range(_B)], axis=0)

    scores_ref[...] = _rows(scores)
    labels_ref[...] = _rows(labels)


def kernel(x, lengths, W1, b1, W2, b2, Wg, bg, Wseg, bseg):
    f32 = jnp.float32
    x2 = x.transpose(0, 2, 1).reshape(_R, _S)
    lrep = jnp.repeat(lengths.astype(jnp.int32), _DIN).reshape(_R, 1)
    len3 = lengths.astype(jnp.int32).reshape(_B, 1, 1)
    b1r = b1.reshape(1, _E)
    b2r = b2.reshape(1, _E)
    bgr = bg.reshape(1, _G)
    bsegr = bseg.reshape(1, _C)

    mean_row, sv_row = pl.pallas_call(
        _stats_kernel,
        out_shape=[
            jax.ShapeDtypeStruct((1, _DIN), f32),
            jax.ShapeDtypeStruct((1, _DIN), f32),
        ],
    )(x2, lrep)

    n2 = _S // _CH2
    gfv = pl.pallas_call(
        _gfv_kernel,
        grid=(n2,),
        in_specs=[
            pl.BlockSpec((_R, _CH2), lambda k: (0, k)),
            pl.BlockSpec((_B, 1, 1), lambda k: (0, 0, 0)),
            pl.BlockSpec((1, _DIN), lambda k: (0, 0)),
            pl.BlockSpec((1, _DIN), lambda k: (0, 0)),
            pl.BlockSpec((_DIN, _E), lambda k: (0, 0)),
            pl.BlockSpec((1, _E), lambda k: (0, 0)),
            pl.BlockSpec((_E, _E), lambda k: (0, 0)),
            pl.BlockSpec((1, _E), lambda k: (0, 0)),
            pl.BlockSpec((_E, _G), lambda k: (0, 0)),
            pl.BlockSpec((1, _G), lambda k: (0, 0)),
        ],
        out_specs=pl.BlockSpec((_B, _G), lambda k: (0, 0)),
        out_shape=jax.ShapeDtypeStruct((_B, _G), f32),
    )(x2, len3, mean_row, sv_row, W1, b1r, W2, b2r, Wg, bgr)

    gfv3 = gfv.reshape(_B, 1, _G)

    n3 = _S // _CH3
    logits, labels, scores, cat = pl.pallas_call(
        _out_kernel,
        grid=(n3,),
        in_specs=[
            pl.BlockSpec((_R, _CH3), lambda k: (0, k)),
            pl.BlockSpec((_B, 1, 1), lambda k: (0, 0, 0)),
            pl.BlockSpec((1, _DIN), lambda k: (0, 0)),
            pl.BlockSpec((1, _DIN), lambda k: (0, 0)),
            pl.BlockSpec((_DIN, _E), lambda k: (0, 0)),
            pl.BlockSpec((1, _E), lambda k: (0, 0)),
            pl.BlockSpec((_E, _E), lambda k: (0, 0)),
            pl.BlockSpec((1, _E), lambda k: (0, 0)),
            pl.BlockSpec((_B, 1, _G), lambda k: (0, 0, 0)),
            pl.BlockSpec((_E + _G, _C), lambda k: (0, 0)),
            pl.BlockSpec((1, _C), lambda k: (0, 0)),
        ],
        out_specs=[
            pl.BlockSpec((_B, _CH3, _C), lambda k: (0, k, 0)),
            pl.BlockSpec((_B, _CH3), lambda k: (0, k)),
            pl.BlockSpec((_B, _CH3), lambda k: (0, k)),
            pl.BlockSpec((_B, _CH3, _E + _G), lambda k: (0, k, 0)),
        ],
        out_shape=[
            jax.ShapeDtypeStruct((_B, _S, _C), f32),
            jax.ShapeDtypeStruct((_B, _S), jnp.int32),
            jax.ShapeDtypeStruct((_B, _S), f32),
            jax.ShapeDtypeStruct((_B, _S, _E + _G), f32),
        ],
    )(x2, len3, mean_row, sv_row, W1, b1r, W2, b2r, gfv3, Wseg, bsegr)

    return (logits, labels[:, :, None], scores[:, :, None], cat)


# speedup vs baseline: 1.4738x; 1.0686x over previous
import jax
import jax.numpy as jnp
from jax.experimental import pallas as pl
from jax.experimental.pallas import tpu as pltpu

_B, _S, _DIN, _E, _G, _C = 16, 4096, 8, 64, 128, 8
_R = _B * _DIN
_CHP = 512
_CH = 256
_NOUT = _S // _CH
_PAD = 0.0


def _rows_from_packed(x2):
    return jnp.concatenate(
        [jnp.transpose(x2[_DIN * b:_DIN * (b + 1), :]) for b in range(_B)],
        axis=0)


def _mlp_h(xs, mean_row, sv_row, w1, b1r, w2, b2r):
    xn = (xs - mean_row) / sv_row
    h = jnp.maximum(xn @ w1 + b1r, 0.0)
    return jnp.maximum(h @ w2 + b2r, 0.0)


def _prep_phase(x2_ref, lrep_ref, len3_ref, w1_ref, b1r_ref, w2_ref,
                b2r_ref, wg_ref, bgr_ref, mean_s, sv_s, gfv_s):
    f32 = jnp.float32
    x2 = x2_ref[...]
    lrep = lrep_ref[...]
    il = jax.lax.broadcasted_iota(jnp.int32, (_R, _S), 1)
    mf = (il < lrep).astype(f32)
    cnt = jnp.maximum(jnp.sum(mf) * (1.0 / _DIN), 1.0)

    rd = jax.lax.broadcasted_iota(jnp.int32, (_DIN, _R), 1)
    dd = jax.lax.broadcasted_iota(jnp.int32, (_DIN, _R), 0)
    m8 = (jax.lax.rem(rd, _DIN) == dd).astype(f32)
    rr = jax.lax.broadcasted_iota(jnp.int32, (_R, _DIN), 0)
    dc = jax.lax.broadcasted_iota(jnp.int32, (_R, _DIN), 1)
    p8 = (jax.lax.rem(rr, _DIN) == dc).astype(f32)

    hi = jax.lax.Precision.HIGHEST
    dn = (((1,), (0,)), ((), ()))
    sum_rows = jnp.sum(x2 * mf, axis=1, keepdims=True)
    mean8 = jax.lax.dot_general(m8, sum_rows, dn, precision=hi) / cnt
    mean_r = jax.lax.dot_general(p8, mean8, dn, precision=hi)
    xc = x2 - mean_r
    sq_rows = jnp.sum((xc * xc) * mf, axis=1, keepdims=True)
    var8 = jax.lax.dot_general(m8, sq_rows, dn, precision=hi) / cnt
    sv8 = jnp.sqrt(var8 + 1e-6)
    sv_r = jax.lax.dot_general(p8, sv8, dn, precision=hi)

    i8r = jax.lax.broadcasted_iota(jnp.int32, (_DIN, _DIN), 0)
    i8c = jax.lax.broadcasted_iota(jnp.int32, (_DIN, _DIN), 1)
    eye8 = (i8r == i8c).astype(f32)
    dt = (((0,), (0,)), ((), ()))
    mean_row = jax.lax.dot_general(mean8, eye8, dt, precision=hi)
    sv_row = jax.lax.dot_general(sv8, eye8, dt, precision=hi)
    mean_s[...] = mean_row
    sv_s[...] = sv_row

    len3 = len3_ref[...]
    w1 = w1_ref[...]
    b1r = b1r_ref[...]
    w2 = w2_ref[...]
    b2r = b2r_ref[...]
    wg = wg_ref[...]
    bgr = bgr_ref[...]
    leng = jnp.broadcast_to(len3, (_B, _CHP, _G))
    igc = jax.lax.broadcasted_iota(jnp.int32, (_B, _CHP, _G), 1)
    gfv = jnp.full((_B, 1, _G), -jnp.inf, dtype=f32)
    for c in range(_S // _CHP):
        xs = _rows_from_packed(x2[:, _CHP * c:_CHP * (c + 1)])
        h = _mlp_h(xs, mean_row, sv_row, w1, b1r, w2, b2r)
        g = jnp.maximum(h @ wg + bgr, 0.0)
        maskg = (igc + _CHP * c) < leng
        g3 = jnp.where(maskg, g.reshape(_B, _CHP, _G), -jnp.inf)
        gfv = jnp.maximum(gfv, jnp.max(g3, axis=1, keepdims=True))
    gfv_s[...] = gfv


def _out_phase(k, x2c_ref, len3_ref, wseg_ref, bsegr_ref, w1_ref, b1r_ref,
               w2_ref, b2r_ref, logits_ref, labels_ref, scores_ref,
               cat_ref, mean_s, sv_s, gfv_s):
    base = (k - 1) * _CH
    len3 = len3_ref[...]

    n = _B * _CH
    h = _mlp_h(_rows_from_packed(x2c_ref[...]), mean_s[...], sv_s[...],
               w1_ref[...], b1r_ref[...], w2_ref[...], b2r_ref[...])

    ie = jax.lax.broadcasted_iota(jnp.int32, (_B, _CH, _E), 1) + base
    maske = ie < jnp.broadcast_to(len3, (_B, _CH, _E))
    h3 = jnp.where(maske, h.reshape(_B, _CH, _E), _PAD)

    ig = jax.lax.broadcasted_iota(jnp.int32, (_B, _CH, _G), 1) + base
    maskg = ig < jnp.broadcast_to(len3, (_B, _CH, _G))
    gfv3 = jnp.broadcast_to(gfv_s[...], (_B, _CH, _G))
    gfv3 = jnp.where(maskg, gfv3, _PAD)

    cat3 = jnp.concatenate([h3, gfv3], axis=2)
    cat_ref[...] = cat3

    logits = cat3.reshape(n, _E + _G) @ wseg_ref[...] + bsegr_ref[...]
    ic = jax.lax.broadcasted_iota(jnp.int32, (_B, _CH, _C), 1) + base
    maskc = ic < jnp.broadcast_to(len3, (_B, _CH, _C))
    logits2 = jnp.where(maskc.reshape(n, _C), logits, _PAD)
    logits_ref[...] = logits2.reshape(_B, _CH, _C)

    lt = jnp.transpose(logits2)
    m = jnp.max(lt, axis=0, keepdims=True)
    ssum = jnp.sum(jnp.exp(lt - m), axis=0, keepdims=True)
    scores = 1.0 / ssum

    cidx = jax.lax.broadcasted_iota(jnp.int32, (_C, n), 0)
    cand = jnp.where(lt == m, cidx, _C)
    labels = jnp.min(cand, axis=0, keepdims=True)
    labels = jnp.where(jnp.isnan(scores), -1, labels)

    def _rows(v):
        return jnp.concatenate(
            [v[:, _CH * b:_CH * (b + 1)] for b in range(_B)], axis=0)

    scores_ref[...] = _rows(scores)
    labels_ref[...] = _rows(labels)


def _fused_kernel(x2_ref, x2c_ref, lrep_ref, len3_ref, w1_ref, b1r_ref,
                  w2_ref, b2r_ref, wg_ref, bgr_ref, wseg_ref, bsegr_ref,
                  logits_ref, labels_ref, scores_ref, cat_ref,
                  mean_s, sv_s, gfv_s):
    k = pl.program_id(0)

    @pl.when(k == 0)
    def _():
        _prep_phase(x2_ref, lrep_ref, len3_ref, w1_ref, b1r_ref, w2_ref,
                    b2r_ref, wg_ref, bgr_ref, mean_s, sv_s, gfv_s)

    @pl.when(k > 0)
    def _():
        _out_phase(k, x2c_ref, len3_ref, wseg_ref, bsegr_ref, w1_ref,
                   b1r_ref, w2_ref, b2r_ref, logits_ref, labels_ref,
                   scores_ref, cat_ref, mean_s, sv_s, gfv_s)


def kernel(x, lengths, W1, b1, W2, b2, Wg, bg, Wseg, bseg):
    f32 = jnp.float32
    x2 = x.transpose(0, 2, 1).reshape(_R, _S)
    lrep = jnp.repeat(lengths.astype(jnp.int32), _DIN).reshape(_R, 1)
    len3 = lengths.astype(jnp.int32).reshape(_B, 1, 1)
    b1r = b1.reshape(1, _E)
    b2r = b2.reshape(1, _E)
    bgr = bg.reshape(1, _G)
    bsegr = bseg.reshape(1, _C)

    def _ochunk(k):
        kk = jnp.maximum(k - 1, 0)
        return kk

    logits, labels, scores, cat = pl.pallas_call(
        _fused_kernel,
        grid=(1 + _NOUT,),
        in_specs=[
            pl.BlockSpec((_R, _S), lambda k: (0, 0)),
            pl.BlockSpec((_R, _CH), lambda k: (0, _ochunk(k))),
            pl.BlockSpec((_R, 1), lambda k: (0, 0)),
            pl.BlockSpec((_B, 1, 1), lambda k: (0, 0, 0)),
            pl.BlockSpec((_DIN, _E), lambda k: (0, 0)),
            pl.BlockSpec((1, _E), lambda k: (0, 0)),
            pl.BlockSpec((_E, _E), lambda k: (0, 0)),
            pl.BlockSpec((1, _E), lambda k: (0, 0)),
            pl.BlockSpec((_E, _G), lambda k: (0, 0)),
            pl.BlockSpec((1, _G), lambda k: (0, 0)),
            pl.BlockSpec((_E + _G, _C), lambda k: (0, 0)),
            pl.BlockSpec((1, _C), lambda k: (0, 0)),
        ],
        out_specs=[
            pl.BlockSpec((_B, _CH, _C), lambda k: (0, _ochunk(k), 0)),
            pl.BlockSpec((_B, _CH), lambda k: (0, _ochunk(k))),
            pl.BlockSpec((_B, _CH), lambda k: (0, _ochunk(k))),
            pl.BlockSpec((_B, _CH, _E + _G), lambda k: (0, _ochunk(k), 0)),
        ],
        out_shape=[
            jax.ShapeDtypeStruct((_B, _S, _C), f32),
            jax.ShapeDtypeStruct((_B, _S), jnp.int32),
            jax.ShapeDtypeStruct((_B, _S), f32),
            jax.ShapeDtypeStruct((_B, _S, _E + _G), f32),
        ],
        scratch_shapes=[
            pltpu.VMEM((1, _DIN), f32),
            pltpu.VMEM((1, _DIN), f32),
            pltpu.VMEM((_B, 1, _G), f32),
        ],
    )(x2, x2, lrep, len3, W1, b1r, W2, b2r, Wg, bgr, Wseg, bsegr)

    return (logits, labels[:, :, None], scores[:, :, None], cat)
